# Initial kernel scaffold; baseline (speedup 1.0000x reference)
import jax
import jax.numpy as jnp
from jax.experimental import pallas as pl


def kernel(x, edge_index, W1, b1, W2, b2):
    raise NotImplementedError("write your pallas kernel here")


# speedup vs baseline: 28.5962x; 28.5962x over previous
import functools

import jax
import jax.numpy as jnp
from jax import lax
from jax.experimental import pallas as pl
from jax.experimental.pallas import tpu as pltpu
from jax.experimental.pallas import tpu_sc as plsc

NC = 2
NS = 16
NW = NC * NS
CH = 128


def _cdiv(a, b):
    return (a + b - 1) // b


def _edge_pass(table, srcb, dstb, zeros):
    T, D = table.shape
    KCH = srcb.shape[1]
    nrows = zeros.shape[0]
    rpt = nrows // NS

    mesh = plsc.VectorSubcoreMesh(
        core_axis_name="c", subcore_axis_name="s", num_cores=NC, num_subcores=NS
    )

    @functools.partial(
        pl.kernel,
        out_type=jax.ShapeDtypeStruct((NC, nrows, D), jnp.float32),
        mesh=mesh,
        scratch_types=[
            pltpu.VMEM((KCH, CH), jnp.int32),
            pltpu.VMEM((KCH, CH), jnp.int32),
            pltpu.VMEM((CH, D), jnp.float32),
            pltpu.VMEM_SHARED((nrows, D), jnp.float32),
            pltpu.SemaphoreType.DMA,
        ],
        compiler_params=pltpu.CompilerParams(use_tc_tiling_on_sc=False),
    )
    def k(table_h, srcb_h, dstb_h, zeros_h, out_h, sidx, didx, rows, acc, sem):
        c = lax.axis_index("c")
        s = lax.axis_index("s")
        wid = c * NS + s
        t0 = s * rpt
        pltpu.sync_copy(zeros_h.at[pl.ds(t0, rpt)], acc.at[pl.ds(t0, rpt)])
        pltpu.sync_copy(srcb_h.at[wid], sidx)
        pltpu.sync_copy(dstb_h.at[wid], didx)
        plsc.subcore_barrier()

        def chunk(j, carry):
            pltpu.async_copy(table_h.at[sidx.at[j]], rows, sem).wait()
            pltpu.sync_copy(rows, acc.at[didx.at[j]], add=True)
            return carry

        lax.fori_loop(0, KCH, chunk, 0)
        plsc.subcore_barrier()
        pltpu.sync_copy(acc.at[pl.ds(t0, rpt)], out_h.at[c, pl.ds(t0, rpt)])

    return k(table, srcb, dstb, zeros)


def _mm1(x, W1):
    N, d_in = x.shape
    hid = W1.shape[1]
    BR = 2000

    def body(x_ref, w_ref, o_ref):
        o_ref[...] = jnp.dot(x_ref[...], w_ref[...],
                             preferred_element_type=jnp.float32)

    return pl.pallas_call(
        body,
        grid=(N // BR,),
        in_specs=[
            pl.BlockSpec((BR, d_in), lambda i: (i, 0)),
            pl.BlockSpec((d_in, hid), lambda i: (0, 0)),
        ],
        out_specs=pl.BlockSpec((BR, hid), lambda i: (i, 0)),
        out_shape=jax.ShapeDtypeStruct((N, hid), jnp.float32),
    )(x, W1)


def _prep(h1, dega, degb):
    N, hid = h1.shape
    BR = 2000

    def body(h_ref, da_ref, db_ref, p_ref, dv_ref):
        dinv = lax.rsqrt(da_ref[...] + db_ref[...] + 1.0)
        dv_ref[...] = dinv
        p_ref[...] = h_ref[...] * dinv

    return pl.pallas_call(
        body,
        grid=(N // BR,),
        in_specs=[
            pl.BlockSpec((BR, hid), lambda i: (i, 0)),
            pl.BlockSpec((BR, 1), lambda i: (i, 0)),
            pl.BlockSpec((BR, 1), lambda i: (i, 0)),
        ],
        out_specs=[
            pl.BlockSpec((BR, hid), lambda i: (i, 0)),
            pl.BlockSpec((BR, 1), lambda i: (i, 0)),
        ],
        out_shape=[
            jax.ShapeDtypeStruct((N, hid), jnp.float32),
            jax.ShapeDtypeStruct((N, 1), jnp.float32),
        ],
    )(h1, dega, degb)


def _mid(s1a, s1b, p, dinv, b1):
    N, hid = p.shape
    BR = 2000

    def body(sa_ref, sb_ref, p_ref, dv_ref, b_ref, q_ref):
        dinv = dv_ref[...]
        a1 = dinv * (sa_ref[...] + sb_ref[...] + p_ref[...]) + b_ref[...]
        h = jnp.where(a1 >= 0, a1, 0.01 * a1)
        q_ref[...] = h * dinv

    return pl.pallas_call(
        body,
        grid=(N // BR,),
        in_specs=[
            pl.BlockSpec((BR, hid), lambda i: (i, 0)),
            pl.BlockSpec((BR, hid), lambda i: (i, 0)),
            pl.BlockSpec((BR, hid), lambda i: (i, 0)),
            pl.BlockSpec((BR, 1), lambda i: (i, 0)),
            pl.BlockSpec((1, hid), lambda i: (0, 0)),
        ],
        out_specs=pl.BlockSpec((BR, hid), lambda i: (i, 0)),
        out_shape=jax.ShapeDtypeStruct((N, hid), jnp.float32),
    )(s1a, s1b, p, dinv, b1.reshape(1, hid))


def _final(s2a, s2b, q, dinv, W2, b2):
    N, hid = q.shape
    d_out = W2.shape[1]
    BR = 2000

    def body(sa_ref, sb_ref, q_ref, dv_ref, w_ref, b_ref, o_ref):
        a2 = dv_ref[...] * (sa_ref[...] + sb_ref[...] + q_ref[...])
        o_ref[...] = jnp.dot(a2, w_ref[...],
                             preferred_element_type=jnp.float32) + b_ref[...]

    return pl.pallas_call(
        body,
        grid=(N // BR,),
        in_specs=[
            pl.BlockSpec((BR, hid), lambda i: (i, 0)),
            pl.BlockSpec((BR, hid), lambda i: (i, 0)),
            pl.BlockSpec((BR, hid), lambda i: (i, 0)),
            pl.BlockSpec((BR, 1), lambda i: (i, 0)),
            pl.BlockSpec((hid, d_out), lambda i: (0, 0)),
            pl.BlockSpec((1, d_out), lambda i: (0, 0)),
        ],
        out_specs=pl.BlockSpec((BR, d_out), lambda i: (i, 0)),
        out_shape=jax.ShapeDtypeStruct((N, d_out), jnp.float32),
    )(s2a, s2b, q, dinv, W2, b2.reshape(1, d_out))


def kernel(x, edge_index, W1, b1, W2, b2):
    N, d_in = x.shape
    E = edge_index.shape[1]
    hid = W1.shape[1]

    KCH = _cdiv(E, NW * CH)
    EP = NW * KCH * CH
    npad = EP - E
    PAD_ROWS = 128
    nrows = _cdiv(N + PAD_ROWS, NS * 8) * NS * 8

    src = edge_index[0]
    dst = edge_index[1]
    ar = jnp.arange(npad, dtype=edge_index.dtype)
    srcb = jnp.concatenate([src, ar % N]).reshape(NW, KCH, CH)
    dstb = jnp.concatenate([dst, N + (ar % PAD_ROWS)]).reshape(NW, KCH, CH)
    zeros = jnp.zeros((nrows, hid), jnp.float32)
    ones_t = jnp.ones((N, hid), jnp.float32)

    degp = _edge_pass(ones_t, srcb, dstb, zeros)
    h1 = _mm1(x, W1)
    p, dinv = _prep(h1, degp[0, :N, 0:1], degp[1, :N, 0:1])
    S1 = _edge_pass(p, srcb, dstb, zeros)
    q = _mid(S1[0, :N], S1[1, :N], p, dinv, b1)
    S2 = _edge_pass(q, srcb, dstb, zeros)
    return _final(S2[0, :N], S2[1, :N], q, dinv, W2, b2)


# speedup vs baseline: 60.5471x; 2.1173x over previous
import functools

import jax
import jax.numpy as jnp
from jax import lax
from jax.experimental import pallas as pl
from jax.experimental.pallas import tpu as pltpu
from jax.experimental.pallas import tpu_sc as plsc

NC = 2
NS = 16
NW = NC * NS
CH = 128


def _cdiv(a, b):
    return (a + b - 1) // b


NBUF = 8


def _edge_pass(table, srcb, dstb, zeros):
    T, D = table.shape
    KCH = srcb.shape[1]
    G = KCH // NBUF
    nrows = zeros.shape[0]
    rpt = nrows // NS

    mesh = plsc.VectorSubcoreMesh(
        core_axis_name="c", subcore_axis_name="s", num_cores=NC, num_subcores=NS
    )

    @functools.partial(
        pl.kernel,
        out_type=jax.ShapeDtypeStruct((NC, nrows, D), jnp.float32),
        mesh=mesh,
        scratch_types=[
            pltpu.VMEM((KCH, CH), jnp.int32),
            pltpu.VMEM((KCH, CH), jnp.int32),
            [pltpu.VMEM((CH, D), jnp.float32) for _ in range(NBUF)],
            pltpu.VMEM_SHARED((nrows, D), jnp.float32),
            pltpu.VMEM_SHARED((T, D), jnp.float32),
            [pltpu.SemaphoreType.DMA for _ in range(NBUF)],
            [pltpu.SemaphoreType.DMA for _ in range(NBUF)],
        ],
        compiler_params=pltpu.CompilerParams(use_tc_tiling_on_sc=False),
    )
    def k(table_h, srcb_h, dstb_h, zeros_h, out_h, sidx, didx, rows, acc,
          tsh, gsem, ssem):
        c = lax.axis_index("c")
        s = lax.axis_index("s")
        wid = c * NS + s
        t0 = s * rpt
        tpt = T // NS

        def gather(j, b):
            return pltpu.make_async_copy(tsh.at[sidx.at[j]], rows[b],
                                         gsem[b])

        def scatter(j, b):
            return pltpu.make_async_copy(rows[b], acc.at[didx.at[j]], ssem[b])

        pltpu.sync_copy(zeros_h.at[pl.ds(t0, rpt)], acc.at[pl.ds(t0, rpt)])
        pltpu.sync_copy(table_h.at[pl.ds(s * tpt, tpt)],
                        tsh.at[pl.ds(s * tpt, tpt)])
        pltpu.sync_copy(srcb_h.at[wid], sidx)
        pltpu.sync_copy(dstb_h.at[wid], didx)
        plsc.subcore_barrier()

        for b in range(NBUF):
            gather(b, b).start()

        def grp(i, carry):
            for b in range(NBUF):
                j = i * NBUF + b
                gather(j, b).wait()
                scatter(j, b).start(add=True)

            @pl.when(i + 1 < G)
            def _():
                for b in range(NBUF):
                    scatter(i * NBUF + b, b).wait()
                    gather((i + 1) * NBUF + b, b).start()
            return carry

        lax.fori_loop(0, G, grp, 0)
        for b in range(NBUF):
            scatter((G - 1) * NBUF + b, b).wait()
        plsc.subcore_barrier()
        pltpu.sync_copy(acc.at[pl.ds(t0, rpt)], out_h.at[c, pl.ds(t0, rpt)])

    return k(table, srcb, dstb, zeros)


def _deg_pass(dstb, ones_v, zeros):
    KCH = dstb.shape[1]
    nrows = zeros.shape[0]
    rpt = nrows // NS
    W = 16

    mesh = plsc.VectorSubcoreMesh(
        core_axis_name="c", subcore_axis_name="s", num_cores=NC, num_subcores=NS
    )

    @functools.partial(
        pl.kernel,
        out_type=jax.ShapeDtypeStruct((NC, nrows, 8), jnp.float32),
        mesh=mesh,
        scratch_types=[
            pltpu.VMEM((KCH, CH), jnp.int32),
            pltpu.VMEM((CH, 8), jnp.float32),
            pltpu.VMEM_SHARED((nrows, 8), jnp.float32),
            pltpu.SemaphoreType.DMA,
        ],
        compiler_params=pltpu.CompilerParams(use_tc_tiling_on_sc=False),
    )
    def k(dstb_h, ones_h, zeros_h, out_h, didx, ones, acc, sem):
        c = lax.axis_index("c")
        s = lax.axis_index("s")
        wid = c * NS + s
        t0 = s * rpt

        def scatter(j):
            return pltpu.make_async_copy(ones, acc.at[didx.at[j]], sem)

        pltpu.sync_copy(zeros_h.at[pl.ds(t0, rpt)], acc.at[pl.ds(t0, rpt)])
        pltpu.sync_copy(dstb_h.at[wid], didx)
        pltpu.sync_copy(ones_h, ones)
        plsc.subcore_barrier()

        def step(j, carry):
            scatter(j).start(add=True)

            @pl.when(j >= W)
            def _():
                scatter(j).wait()
            return carry

        lax.fori_loop(0, KCH, step, 0)

        def drain(j, carry):
            scatter(j).wait()
            return carry

        lax.fori_loop(0, min(W, KCH), drain, 0)
        plsc.subcore_barrier()
        pltpu.sync_copy(acc.at[pl.ds(t0, rpt)], out_h.at[c, pl.ds(t0, rpt)])

    return k(dstb, ones_v, zeros)


def _mm1p(x, W1, dega, degb):
    N, d_in = x.shape
    hid = W1.shape[1]
    BR = 2000

    def body(x_ref, w_ref, da_ref, db_ref, p_ref, dv_ref):
        h1 = jnp.dot(x_ref[...], w_ref[...], preferred_element_type=jnp.float32)
        dinv = lax.rsqrt(da_ref[...] + db_ref[...] + 1.0)
        dv_ref[...] = dinv
        p_ref[...] = h1 * dinv

    return pl.pallas_call(
        body,
        grid=(N // BR,),
        in_specs=[
            pl.BlockSpec((BR, d_in), lambda i: (i, 0)),
            pl.BlockSpec((d_in, hid), lambda i: (0, 0)),
            pl.BlockSpec((BR, 1), lambda i: (i, 0)),
            pl.BlockSpec((BR, 1), lambda i: (i, 0)),
        ],
        out_specs=[
            pl.BlockSpec((BR, hid), lambda i: (i, 0)),
            pl.BlockSpec((BR, 1), lambda i: (i, 0)),
        ],
        out_shape=[
            jax.ShapeDtypeStruct((N, hid), jnp.float32),
            jax.ShapeDtypeStruct((N, 1), jnp.float32),
        ],
    )(x, W1, dega, degb)


def _mid(s1a, s1b, p, dinv, b1):
    N, hid = p.shape
    BR = 2000

    def body(sa_ref, sb_ref, p_ref, dv_ref, b_ref, q_ref):
        dinv = dv_ref[...]
        a1 = dinv * (sa_ref[...] + sb_ref[...] + p_ref[...]) + b_ref[...]
        h = jnp.where(a1 >= 0, a1, 0.01 * a1)
        q_ref[...] = h * dinv

    return pl.pallas_call(
        body,
        grid=(N // BR,),
        in_specs=[
            pl.BlockSpec((BR, hid), lambda i: (i, 0)),
            pl.BlockSpec((BR, hid), lambda i: (i, 0)),
            pl.BlockSpec((BR, hid), lambda i: (i, 0)),
            pl.BlockSpec((BR, 1), lambda i: (i, 0)),
            pl.BlockSpec((1, hid), lambda i: (0, 0)),
        ],
        out_specs=pl.BlockSpec((BR, hid), lambda i: (i, 0)),
        out_shape=jax.ShapeDtypeStruct((N, hid), jnp.float32),
    )(s1a, s1b, p, dinv, b1.reshape(1, hid))


def _final(s2a, s2b, q, dinv, W2, b2):
    N, hid = q.shape
    d_out = W2.shape[1]
    BR = 2000

    def body(sa_ref, sb_ref, q_ref, dv_ref, w_ref, b_ref, o_ref):
        a2 = dv_ref[...] * (sa_ref[...] + sb_ref[...] + q_ref[...])
        o_ref[...] = jnp.dot(a2, w_ref[...],
                             preferred_element_type=jnp.float32) + b_ref[...]

    return pl.pallas_call(
        body,
        grid=(N // BR,),
        in_specs=[
            pl.BlockSpec((BR, hid), lambda i: (i, 0)),
            pl.BlockSpec((BR, hid), lambda i: (i, 0)),
            pl.BlockSpec((BR, hid), lambda i: (i, 0)),
            pl.BlockSpec((BR, 1), lambda i: (i, 0)),
            pl.BlockSpec((hid, d_out), lambda i: (0, 0)),
            pl.BlockSpec((1, d_out), lambda i: (0, 0)),
        ],
        out_specs=pl.BlockSpec((BR, d_out), lambda i: (i, 0)),
        out_shape=jax.ShapeDtypeStruct((N, d_out), jnp.float32),
    )(s2a, s2b, q, dinv, W2, b2.reshape(1, d_out))


def kernel(x, edge_index, W1, b1, W2, b2):
    N, d_in = x.shape
    E = edge_index.shape[1]
    hid = W1.shape[1]

    KCH = _cdiv(_cdiv(E, NW * CH), NBUF) * NBUF
    EP = NW * KCH * CH
    npad = EP - E
    PAD_ROWS = 128
    nrows = _cdiv(N + PAD_ROWS, NS * 8) * NS * 8

    src = edge_index[0]
    dst = edge_index[1]
    ar = jnp.arange(npad, dtype=edge_index.dtype)
    srcb = jnp.concatenate([src, ar % N]).reshape(NW, KCH, CH)
    dstb = jnp.concatenate([dst, N + (ar % PAD_ROWS)]).reshape(NW, KCH, CH)
    zeros = jnp.zeros((nrows, hid), jnp.float32)
    zeros1 = jnp.zeros((nrows, 8), jnp.float32)
    ones_v = jnp.ones((CH, 8), jnp.float32)

    degp = _deg_pass(dstb, ones_v, zeros1)
    p, dinv = _mm1p(x, W1, degp[0, :N, 0:1], degp[1, :N, 0:1])
    S1 = _edge_pass(p, srcb, dstb, zeros)
    q = _mid(S1[0, :N], S1[1, :N], p, dinv, b1)
    S2 = _edge_pass(q, srcb, dstb, zeros)
    return _final(S2[0, :N], S2[1, :N], q, dinv, W2, b2)
